# baseline (speedup 1.0000x reference)
import jax
import jax.numpy as jnp
from jax import lax
from jax.experimental import pallas as pl
from jax.experimental.pallas import tpu as pltpu
from jax.experimental.pallas import tpu_sc as plsc

B = 16384
A = 200
NC = 2
NS = 16
NW = NC * NS
R = B // NW
L = 16
TR = B // 128
CHI = A // 8
KPW = R // 128


def _sc_body(spec_hbm, energies_hbm, table_hbm, out_hbm,
             buf0, buf1, en_v, tab_v, res_v, sem0, sem1):
    wid = lax.axis_index("s") * NC + lax.axis_index("c")
    base = wid * R

    pltpu.sync_copy(energies_hbm.at[pl.ds(base, R)], en_v)
    pltpu.sync_copy(table_hbm, tab_v.at[pl.ds(0, 7)])

    bufs = (buf0, buf1)
    sems = (sem0, sem1)

    def start(k):
        return pltpu.async_copy(
            spec_hbm.at[:, wid * KPW + k, :], bufs[k % 2], sems[k % 2])

    t_vec = tab_v[...]

    copies = [start(0)]
    for k in range(KPW):
        if k + 1 < KPW:
            copies.append(start(k + 1))
        copies[k].wait()
        buf = bufs[k % 2]

        def lane_group(g, _):
            g16 = g * L

            def col_tile(chi, carry):
                acc0, acc1 = carry
                for clo in range(8):
                    sv = buf[chi, pl.ds(clo * 128 + g16, L)]
                    sae = jnp.take_along_axis(t_vec, sv, axis=0,
                                              mode="promise_in_bounds")
                    if clo % 2 == 0:
                        acc0 = acc0 + sae
                    else:
                        acc1 = acc1 + sae
                return acc0, acc1

            z = jnp.zeros((L,), jnp.float32)
            acc0, acc1 = lax.fori_loop(0, CHI, col_tile, (z, z))
            rbase = k * 128 + g16
            res_v[pl.ds(rbase, L)] = (acc0 + acc1) + en_v[pl.ds(rbase, L)]
            return 0

        lax.fori_loop(0, 128 // L, lane_group, 0)

    pltpu.sync_copy(res_v, out_hbm.at[pl.ds(base, R)])


def _tc_copy_body(x_ref, o_ref):
    o_ref[...] = x_ref[...]


@jax.jit
def _shift(spec_lin, energies, self_energies):
    mesh = plsc.VectorSubcoreMesh(core_axis_name="c", subcore_axis_name="s")
    fn = pl.kernel(
        _sc_body,
        mesh=mesh,
        compiler_params=pltpu.CompilerParams(use_tc_tiling_on_sc=False,
                                             needs_layout_passes=False),
        out_type=jax.ShapeDtypeStruct((B,), jnp.float32),
        scratch_types=[
            pltpu.VMEM((CHI, 1024), jnp.int32),
            pltpu.VMEM((CHI, 1024), jnp.int32),
            pltpu.VMEM((R,), jnp.float32),
            pltpu.VMEM((L,), jnp.float32),
            pltpu.VMEM((R,), jnp.float32),
            pltpu.SemaphoreType.DMA,
            pltpu.SemaphoreType.DMA,
        ],
    )
    return fn(spec_lin, energies, self_energies)


@jax.jit
def _passthrough(species_t):
    return pl.pallas_call(
        _tc_copy_body,
        grid=(CHI,),
        in_specs=[pl.BlockSpec((8, B), lambda i: (i, 0))],
        out_specs=pl.BlockSpec((8, B), lambda i: (i, 0)),
        out_shape=jax.ShapeDtypeStruct((A, B), jnp.int32),
    )(species_t)


def kernel(species, energies, self_energies):
    spec_lin = (species.astype(jnp.int32)
                .reshape(TR, 128, CHI, 8)
                .transpose(2, 0, 3, 1)
                .reshape(CHI, TR, 1024))
    out = _shift(spec_lin, energies, self_energies)
    species_out = _passthrough(species.astype(jnp.int32).T).T
    return (species_out.astype(species.dtype), out)
